# baseline (speedup 1.0000x reference)
import functools

import jax
import jax.numpy as jnp
from jax import lax
from jax.experimental import pallas as pl
from jax.experimental.pallas import tpu as pltpu
from jax.experimental.pallas import tpu_sc as plsc

VOCAB = 1000000
EMBED_DIM = 32
BATCH = 16384

NUM_CORES = 2
NUM_SUBCORES = 16
NUM_WORKERS = NUM_CORES * NUM_SUBCORES
LANES = 16

B_PER_W = BATCH // NUM_WORKERS
N_CHUNK = 4
CHUNK = B_PER_W // N_CHUNK
GROUPS = B_PER_W // LANES


def _rsqrt16(x):
    i = lax.bitcast_convert_type(x, jnp.int32)
    i = 0x5F3759DF - lax.shift_right_logical(i, 1)
    y = lax.bitcast_convert_type(i, jnp.float32)
    for _ in range(3):
        y = y * (1.5 - 0.5 * x * y * y)
    return y


def _tower_body(ids_hbm, table_hbm, out_hbm, idx_v, rows_v, sem):
    wid = lax.axis_index("s") * NUM_CORES + lax.axis_index("c")

    pltpu.sync_copy(ids_hbm.at[wid], idx_v)
    copies = [
        pltpu.async_copy(
            table_hbm.at[idx_v.at[j]],
            rows_v.at[pl.ds(j * CHUNK, CHUNK)],
            sem,
        )
        for j in range(N_CHUNK)
    ]
    for cp in copies:
        cp.wait()

    lanes = lax.iota(jnp.int32, LANES)

    def group(g, carry):
        row_idx = g * LANES + lanes
        acc = jnp.zeros((LANES,), jnp.float32)
        cols = []
        for j in range(EMBED_DIM):
            col = plsc.load_gather(
                rows_v, [row_idx, jnp.full((LANES,), j, jnp.int32)]
            )
            cols.append(col)
            acc = acc + col * col
        scale = _rsqrt16(jnp.maximum(acc, 1e-24))
        for j in range(EMBED_DIM):
            plsc.store_scatter(
                rows_v,
                [row_idx, jnp.full((LANES,), j, jnp.int32)],
                cols[j] * scale,
            )
        return carry

    lax.fori_loop(0, GROUPS, group, 0)

    base = wid * B_PER_W
    pltpu.sync_copy(rows_v, out_hbm.at[pl.ds(base, B_PER_W)])


_tower = functools.partial(
    pl.kernel,
    out_type=jax.ShapeDtypeStruct((BATCH, EMBED_DIM), jnp.float32),
    mesh=plsc.VectorSubcoreMesh(core_axis_name="c", subcore_axis_name="s"),
    compiler_params=pltpu.CompilerParams(
        use_tc_tiling_on_sc=False, needs_layout_passes=False
    ),
    scratch_types=[
        pltpu.VMEM((N_CHUNK, CHUNK), jnp.int32),
        pltpu.VMEM((B_PER_W, EMBED_DIM), jnp.float32),
        pltpu.SemaphoreType.DMA,
    ],
)(_tower_body)


def kernel(item_ids, embedding_table):
    ids = item_ids.astype(jnp.int32).reshape(NUM_WORKERS, N_CHUNK, CHUNK)
    return _tower(ids, embedding_table)


# speedup vs baseline: 1.5711x; 1.5711x over previous
import functools

import jax
import jax.numpy as jnp
from jax import lax
from jax.experimental import pallas as pl
from jax.experimental.pallas import tpu as pltpu
from jax.experimental.pallas import tpu_sc as plsc

VOCAB = 1000000
EMBED_DIM = 32
BATCH = 16384

NUM_CORES = 2
NUM_SUBCORES = 16
NUM_WORKERS = NUM_CORES * NUM_SUBCORES
LANES = 16

B_PER_W = BATCH // NUM_WORKERS
CHUNK = 16
N_CHUNK = B_PER_W // CHUNK
NBUF = 2


def _scalar_rsqrt(x):
    i = lax.bitcast_convert_type(x, jnp.int32)
    i = 0x5F3759DF - lax.shift_right_logical(i, 1)
    y = lax.bitcast_convert_type(i, jnp.float32)
    for _ in range(3):
        y = y * (1.5 - 0.5 * x * y * y)
    return y


def _tower_body(ids_hbm, table_hbm, out_hbm, idsv, tidv, subv, tiles, rows, sem):
    wid = lax.axis_index("s") * NUM_CORES + lax.axis_index("c")

    pltpu.sync_copy(ids_hbm.at[pl.ds(wid * 4, 4)], idsv)

    for j in range(4):
        for k in range(8):
            v = idsv[j, pl.ds(k * LANES, LANES)]
            base = j * 128 + k * LANES
            tidv[pl.ds(base, LANES)] = lax.shift_right_logical(v, 3)
            subv[pl.ds(base, LANES)] = lax.bitwise_and(v, 7)

    def fetch(c, slot):
        tids = tidv[pl.ds(c * CHUNK, LANES)]
        subs = subv[pl.ds(c * CHUNK, LANES)]
        for r in range(CHUNK):
            tid = lax.squeeze(lax.slice(tids, (r,), (r + 1,)), (0,))
            sub = lax.squeeze(lax.slice(subs, (r,), (r + 1,)), (0,))
            pltpu.async_copy(
                table_hbm.at[tid, sub],
                tiles.at[slot, r],
                sem,
            )

    fetch(0, 0)

    def step(c, carry):
        slot = lax.rem(c, NBUF)
        @pl.when(c + 1 < N_CHUNK)
        def _():
            fetch(c + 1, lax.rem(c + 1, NBUF))

        for r in range(CHUNK):
            pltpu.make_async_copy(
                table_hbm.at[0, 0], tiles.at[slot, r], sem
            ).wait()

        for r in range(CHUNK):
            a = tiles[slot, r, pl.ds(0, LANES)]
            b = tiles[slot, r, pl.ds(LANES, LANES)]
            h = a * a + b * b
            ssq = jnp.sum(h)
            scale = _scalar_rsqrt(jnp.maximum(ssq, 1e-24))
            row = c * CHUNK + r
            t, s = row // 8, row % 8
            rows[t, s, pl.ds(0, LANES)] = a * scale
            rows[t, s, pl.ds(LANES, LANES)] = b * scale
        return carry

    lax.fori_loop(0, N_CHUNK, step, 0, unroll=False)

    pltpu.sync_copy(rows, out_hbm.at[pl.ds(wid * (B_PER_W // 8), B_PER_W // 8)])


_tower = functools.partial(
    pl.kernel,
    out_type=jax.ShapeDtypeStruct((BATCH // 8, 8, EMBED_DIM), jnp.float32),
    mesh=plsc.VectorSubcoreMesh(core_axis_name="c", subcore_axis_name="s"),
    compiler_params=pltpu.CompilerParams(needs_layout_passes=False),
    scratch_types=[
        pltpu.VMEM((4, 128), jnp.int32),
        pltpu.VMEM((B_PER_W,), jnp.int32),
        pltpu.VMEM((B_PER_W,), jnp.int32),
        pltpu.VMEM((NBUF, CHUNK, EMBED_DIM), jnp.float32),
        pltpu.VMEM((B_PER_W // 8, 8, EMBED_DIM), jnp.float32),
        pltpu.SemaphoreType.DMA,
    ],
)(_tower_body)


def kernel(item_ids, embedding_table):
    ids = item_ids.astype(jnp.int32).reshape(128, 128)
    table3 = embedding_table.reshape(VOCAB // 8, 8, EMBED_DIM)
    out3 = _tower(ids, table3)
    return out3.reshape(BATCH, EMBED_DIM)
